# Initial kernel scaffold; baseline (speedup 1.0000x reference)
import jax
import jax.numpy as jnp
from jax.experimental import pallas as pl


def kernel(seq_len, emb):
    raise NotImplementedError("write your pallas kernel here")


# speedup vs baseline: 36.4248x; 36.4248x over previous
import functools

import jax
import jax.numpy as jnp
from jax import lax
from jax.experimental import pallas as pl
from jax.experimental.pallas import tpu as pltpu
from jax.experimental.pallas import tpu_sc as plsc

_NUM_HEADS = 16
_R = 128
_S = 2048
_TAB = 2 * _R + 1
_G = 16
_WCOLS = 4096
_NW = 32
_ROWS_PER_W = _S * _NUM_HEADS // _NW

_mesh = plsc.VectorSubcoreMesh(core_axis_name="c", subcore_axis_name="s")


@functools.partial(
    pl.kernel,
    mesh=_mesh,
    out_type=jax.ShapeDtypeStruct((_NUM_HEADS, _S, _S), jnp.float32),
    compiler_params=pltpu.CompilerParams(
        needs_layout_passes=False, use_tc_tiling_on_sc=False
    ),
    scratch_types=(
        [pltpu.VMEM((_TAB * _NUM_HEADS,), jnp.float32)]
        + [pltpu.VMEM((_WCOLS,), jnp.float32)] * _G
    ),
)
def _rpe_sc(emb_hbm, out_hbm, emb_v, *w_refs):
    wid = lax.axis_index("s") * 2 + lax.axis_index("c")
    h = wid % _NUM_HEADS
    i0 = (wid // _NUM_HEADS) * _ROWS_PER_W

    pltpu.sync_copy(emb_hbm, emb_v)

    lanes = lax.iota(jnp.int32, 16)
    hvec = jnp.full((16,), h, dtype=jnp.int32)

    def build(c, carry):
        base = c * 16
        for r in range(_G):
            m = base + lanes + r
            idx = jnp.clip(m - (_S - 1 - _R), 0, _TAB - 1) * _NUM_HEADS + hvec
            w_refs[r][pl.ds(base, 16)] = plsc.load_gather(emb_v, [idx])
        return carry

    lax.fori_loop(0, _WCOLS // 16, build, 0)

    for r in range(_G):
        off = (_S - 1 - r) % _G

        def emit(t, carry, r=r, off=off):
            i = i0 + off + 16 * t
            q = pl.multiple_of((_S - 1 - r) - i, _G)
            pltpu.sync_copy(w_refs[r].at[pl.ds(q, _S)], out_hbm.at[h, i])
            return carry

        lax.fori_loop(0, _ROWS_PER_W // _G, emit, 0)


def kernel(seq_len, emb):
    del seq_len
    return _rpe_sc(emb.reshape(_TAB * _NUM_HEADS))


# speedup vs baseline: 41.8355x; 1.1485x over previous
import functools

import jax
import jax.numpy as jnp
from jax import lax
from jax.experimental import pallas as pl
from jax.experimental.pallas import tpu as pltpu
from jax.experimental.pallas import tpu_sc as plsc

_NUM_HEADS = 16
_R = 128
_S = 2048
_TAB = 2 * _R + 1
_G = 16
_WCOLS = 4096
_NW = 32
_ROWS_PER_W = _S * _NUM_HEADS // _NW

_mesh = plsc.VectorSubcoreMesh(core_axis_name="c", subcore_axis_name="s")


@functools.partial(
    pl.kernel,
    mesh=_mesh,
    out_type=jax.ShapeDtypeStruct((_NUM_HEADS, _S, _S), jnp.float32),
    compiler_params=pltpu.CompilerParams(
        needs_layout_passes=False, use_tc_tiling_on_sc=False
    ),
    scratch_types=(
        [pltpu.VMEM((_TAB * _NUM_HEADS,), jnp.float32)]
        + [pltpu.VMEM((_WCOLS,), jnp.float32)] * _G
        + [pltpu.SemaphoreType.DMA]
    ),
)
def _rpe_sc(emb_hbm, out_hbm, emb_v, *scratch):
    w_refs, sem = scratch[:_G], scratch[_G]
    wid = lax.axis_index("s") * 2 + lax.axis_index("c")
    h = wid % _NUM_HEADS
    i0 = (wid // _NUM_HEADS) * _ROWS_PER_W

    pltpu.sync_copy(emb_hbm, emb_v)

    lanes = lax.iota(jnp.int32, 16)
    hvec = jnp.full((16,), h, dtype=jnp.int32)

    def build(c, carry):
        base = c * 16
        for r in range(_G):
            m = base + lanes + r
            idx = jnp.clip(m - (_S - 1 - _R), 0, _TAB - 1) * _NUM_HEADS + hvec
            w_refs[r][pl.ds(base, 16)] = plsc.load_gather(emb_v, [idx])
        return carry

    lax.fori_loop(0, _WCOLS // 16, build, 0)

    def emit(t, carry):
        copies = []
        for r in range(_G):
            off = (_S - 1 - r) % _G
            i = i0 + off + 16 * t
            q = pl.multiple_of((_S - 1 - r) - i, _G)
            c = pltpu.make_async_copy(
                w_refs[r].at[pl.ds(q, _S)], out_hbm.at[h, i], sem
            )
            c.start()
            copies.append(c)
        for c in copies:
            c.wait()
        return carry

    lax.fori_loop(0, _ROWS_PER_W // _G, emit, 0)


def kernel(seq_len, emb):
    del seq_len
    return _rpe_sc(emb.reshape(_TAB * _NUM_HEADS))


# speedup vs baseline: 79.3938x; 1.8978x over previous
import functools

import jax
import jax.numpy as jnp
from jax import lax
from jax.experimental import pallas as pl
from jax.experimental.pallas import tpu as pltpu
from jax.experimental.pallas import tpu_sc as plsc

_NUM_HEADS = 16
_R = 128
_S = 2048
_TAB = 2 * _R + 1
_G = 16
_WCOLS = 4096
_BAND = 512
_NBLK = _S // 8
_NGRP = _NBLK // 16

_mesh = plsc.VectorSubcoreMesh(core_axis_name="c", subcore_axis_name="s")


@functools.partial(
    pl.kernel,
    mesh=_mesh,
    out_type=jax.ShapeDtypeStruct((_NUM_HEADS, _S, _S), jnp.float32),
    compiler_params=pltpu.CompilerParams(needs_layout_passes=False),
    scratch_types=(
        [pltpu.VMEM((_TAB * _NUM_HEADS,), jnp.float32)]
        + [pltpu.VMEM((_WCOLS,), jnp.float32)] * _G
        + [
            pltpu.VMEM((8, _S), jnp.float32),
            pltpu.VMEM((8, _S), jnp.float32),
            pltpu.VMEM((8, _BAND), jnp.float32),
            pltpu.VMEM((8, _BAND), jnp.float32),
            pltpu.SemaphoreType.DMA,
            pltpu.SemaphoreType.DMA,
        ]
    ),
)
def _rpe_sc(emb_hbm, out_hbm, emb_v, *scratch):
    w_refs = scratch[:_G]
    lo_v, hi_v, band0, band1, sem_b, sem_c = scratch[_G:]
    wid = lax.axis_index("s") * 2 + lax.axis_index("c")
    h = wid % _NUM_HEADS
    half = wid // _NUM_HEADS

    pltpu.sync_copy(emb_hbm, emb_v)

    lanes = lax.iota(jnp.int32, 16)
    hvec = jnp.full((16,), h, dtype=jnp.int32)

    def build_w(c, carry):
        base = c * 16
        for r in range(_G):
            m = base + lanes + r
            idx = jnp.clip(m - (_S - 1 - _R), 0, _TAB - 1) * _NUM_HEADS + hvec
            w_refs[r][pl.ds(base, 16)] = plsc.load_gather(emb_v, [idx])
        return carry

    lax.fori_loop(0, _WCOLS // 16, build_w, 0)

    lo_vec = plsc.load_gather(emb_v, [hvec])
    hi_vec = plsc.load_gather(emb_v, [(_TAB - 1) * _NUM_HEADS + hvec])

    def fill_const(c, carry):
        for r_lo in range(8):
            lo_v[r_lo, pl.ds(c * 16, 16)] = lo_vec
            hi_v[r_lo, pl.ds(c * 16, 16)] = hi_vec
        return carry

    lax.fori_loop(0, _S // 16, fill_const, 0)

    def make_band_builder(band_ref, b):
        def build_band(blk, col0):
            s_base = _S - 1 - 8 * blk

            def chunk(cc, carry):
                for r_lo in range(8):
                    r = (15 - 8 * b - r_lo) % _G
                    q = pl.multiple_of(s_base - r_lo - r, _G)
                    src = pl.multiple_of(q + col0 + cc * 16, _G)
                    band_ref[r_lo, pl.ds(cc * 16, 16)] = w_refs[r][pl.ds(src, 16)]
                return carry

            lax.fori_loop(0, _BAND // 16, chunk, 0)

        return build_band

    builders = (make_band_builder(band0, 0), make_band_builder(band1, 1))
    bands = (band0, band1)

    for g in range(_NGRP):
        tb = min(12, max(0, g - 1))
        left = 128 * tb
        right0 = left + _BAND
        rlen = _S - right0

        @pl.when((g // (_NGRP // 2)) == half)
        def _(g=g, tb=tb, left=left, right0=right0, rlen=rlen):
            def pair(tt2, carry):
                for b in range(2):
                    blk = g * 16 + tt2 * 2 + b
                    i_b = pl.multiple_of(8 * blk, 8)
                    builders[b](blk, left)
                    if left > 0:
                        pltpu.make_async_copy(
                            lo_v.at[:, pl.ds(0, left)],
                            out_hbm.at[h, pl.ds(i_b, 8), pl.ds(0, left)],
                            sem_c,
                        ).start()
                    pltpu.make_async_copy(
                        bands[b],
                        out_hbm.at[h, pl.ds(i_b, 8), pl.ds(left, _BAND)],
                        sem_b,
                    ).start()
                    if rlen > 0:
                        pltpu.make_async_copy(
                            hi_v.at[:, pl.ds(0, rlen)],
                            out_hbm.at[h, pl.ds(i_b, 8), pl.ds(right0, rlen)],
                            sem_c,
                        ).start()
                for b in range(2):
                    blk = g * 16 + tt2 * 2 + b
                    i_b = pl.multiple_of(8 * blk, 8)
                    pltpu.make_async_copy(
                        bands[b],
                        out_hbm.at[h, pl.ds(i_b, 8), pl.ds(left, _BAND)],
                        sem_b,
                    ).wait()
                return carry

            lax.fori_loop(0, 8, pair, 0)

            def drain(tt, carry):
                i_b = pl.multiple_of(8 * (g * 16 + tt), 8)
                if left > 0:
                    pltpu.make_async_copy(
                        lo_v.at[:, pl.ds(0, left)],
                        out_hbm.at[h, pl.ds(i_b, 8), pl.ds(0, left)],
                        sem_c,
                    ).wait()
                if rlen > 0:
                    pltpu.make_async_copy(
                        hi_v.at[:, pl.ds(0, rlen)],
                        out_hbm.at[h, pl.ds(i_b, 8), pl.ds(right0, rlen)],
                        sem_c,
                    ).wait()
                return carry

            if left > 0 or rlen > 0:
                lax.fori_loop(0, 16, drain, 0)


def kernel(seq_len, emb):
    del seq_len
    return _rpe_sc(emb.reshape(_TAB * _NUM_HEADS))


# speedup vs baseline: 85.8884x; 1.0818x over previous
import functools

import jax
import jax.numpy as jnp
from jax import lax
from jax.experimental import pallas as pl
from jax.experimental.pallas import tpu as pltpu
from jax.experimental.pallas import tpu_sc as plsc

_NUM_HEADS = 16
_R = 128
_S = 2048
_TAB = 2 * _R + 1
_G = 16
_WCOLS = 4096
_BAND = 512
_NBLK = _S // 8
_NGRP = _NBLK // 16

_mesh = plsc.VectorSubcoreMesh(core_axis_name="c", subcore_axis_name="s")


@functools.partial(
    pl.kernel,
    mesh=_mesh,
    out_type=jax.ShapeDtypeStruct((_NUM_HEADS, _S, _S), jnp.float32),
    compiler_params=pltpu.CompilerParams(needs_layout_passes=False),
    scratch_types=(
        [pltpu.VMEM((_TAB * _NUM_HEADS,), jnp.float32)]
        + [pltpu.VMEM((_WCOLS,), jnp.float32)] * _G
        + [
            pltpu.VMEM((8, _S), jnp.float32),
            pltpu.VMEM((8, _S), jnp.float32),
            pltpu.VMEM((8, _BAND), jnp.float32),
            pltpu.VMEM((8, _BAND), jnp.float32),
            pltpu.VMEM((8, _BAND), jnp.float32),
            pltpu.VMEM((8, _BAND), jnp.float32),
            pltpu.SemaphoreType.DMA,
            pltpu.SemaphoreType.DMA,
        ]
    ),
)
def _rpe_sc(emb_hbm, out_hbm, emb_v, *scratch):
    w_refs = scratch[:_G]
    lo_v, hi_v = scratch[_G], scratch[_G + 1]
    bands = scratch[_G + 2:_G + 6]
    sem_b, sem_c = scratch[_G + 6], scratch[_G + 7]
    wid = lax.axis_index("s") * 2 + lax.axis_index("c")
    h = wid % _NUM_HEADS
    half = wid // _NUM_HEADS

    pltpu.sync_copy(emb_hbm, emb_v)

    lanes = lax.iota(jnp.int32, 16)
    hvec = jnp.full((16,), h, dtype=jnp.int32)

    def build_w(c, carry):
        base = c * 16
        for r in range(_G):
            m = base + lanes + r
            idx = jnp.clip(m - (_S - 1 - _R), 0, _TAB - 1) * _NUM_HEADS + hvec
            w_refs[r][pl.ds(base, 16)] = plsc.load_gather(emb_v, [idx])
        return carry

    lax.fori_loop(0, _WCOLS // 16, build_w, 0)

    lo_vec = plsc.load_gather(emb_v, [hvec])
    hi_vec = plsc.load_gather(emb_v, [(_TAB - 1) * _NUM_HEADS + hvec])

    def fill_const(c, carry):
        for r_lo in range(8):
            lo_v[r_lo, pl.ds(c * 16, 16)] = lo_vec
            hi_v[r_lo, pl.ds(c * 16, 16)] = hi_vec
        return carry

    lax.fori_loop(0, _S // 16, fill_const, 0)

    def make_band_builder(band_ref, b):
        def build_band(blk, col0):
            s_base = _S - 1 - 8 * blk

            def chunk(cc, carry):
                for r_lo in range(8):
                    r = (15 - 8 * b - r_lo) % _G
                    q = pl.multiple_of(s_base - r_lo - r, _G)
                    src = pl.multiple_of(q + col0 + cc * 16, _G)
                    band_ref[r_lo, pl.ds(cc * 16, 16)] = w_refs[r][pl.ds(src, 16)]
                return carry

            lax.fori_loop(0, _BAND // 16, chunk, 0)

        return build_band

    builders = tuple(
        make_band_builder(bands[b], b % 2) for b in range(4)
    )

    for g in range(_NGRP):
        tb = min(12, max(0, g - 1))
        left = 128 * tb
        right0 = left + _BAND
        rlen = _S - right0

        @pl.when((g // (_NGRP // 2)) == half)
        def _(g=g, tb=tb, left=left, right0=right0, rlen=rlen):
            def quad(qq, carry):
                for b in range(4):
                    blk = g * 16 + qq * 4 + b
                    i_b = pl.multiple_of(8 * blk, 8)
                    if g % (_NGRP // 2) == 0:
                        @pl.when(qq > 0)
                        def _():
                            pltpu.make_async_copy(
                                bands[b],
                                out_hbm.at[h, pl.ds(0, 8), pl.ds(left, _BAND)],
                                sem_b,
                            ).wait()
                    else:
                        pltpu.make_async_copy(
                            bands[b],
                            out_hbm.at[h, pl.ds(0, 8), pl.ds(left, _BAND)],
                            sem_b,
                        ).wait()
                    builders[b](blk, left)
                    if left > 0:
                        pltpu.make_async_copy(
                            lo_v.at[:, pl.ds(0, left)],
                            out_hbm.at[h, pl.ds(i_b, 8), pl.ds(0, left)],
                            sem_c,
                        ).start()
                    pltpu.make_async_copy(
                        bands[b],
                        out_hbm.at[h, pl.ds(i_b, 8), pl.ds(left, _BAND)],
                        sem_b,
                    ).start()
                    if rlen > 0:
                        pltpu.make_async_copy(
                            hi_v.at[:, pl.ds(0, rlen)],
                            out_hbm.at[h, pl.ds(i_b, 8), pl.ds(right0, rlen)],
                            sem_c,
                        ).start()
                return carry

            lax.fori_loop(0, 4, quad, 0)

            def drain(tt, carry):
                i_b = pl.multiple_of(8 * (g * 16 + tt), 8)
                if left > 0:
                    pltpu.make_async_copy(
                        lo_v.at[:, pl.ds(0, left)],
                        out_hbm.at[h, pl.ds(i_b, 8), pl.ds(0, left)],
                        sem_c,
                    ).wait()
                if rlen > 0:
                    pltpu.make_async_copy(
                        hi_v.at[:, pl.ds(0, rlen)],
                        out_hbm.at[h, pl.ds(i_b, 8), pl.ds(right0, rlen)],
                        sem_c,
                    ).wait()
                return carry

            if left > 0 or rlen > 0:
                lax.fori_loop(0, 16, drain, 0)

    for b in range(4):
        pltpu.make_async_copy(
            bands[b], out_hbm.at[h, pl.ds(0, 8), pl.ds(0, _BAND)], sem_b
        ).wait()


def kernel(seq_len, emb):
    del seq_len
    return _rpe_sc(emb.reshape(_TAB * _NUM_HEADS))


# speedup vs baseline: 120.0214x; 1.3974x over previous
import functools

import jax
import jax.numpy as jnp
from jax import lax
from jax.experimental import pallas as pl
from jax.experimental.pallas import tpu as pltpu
from jax.experimental.pallas import tpu_sc as plsc

_NUM_HEADS = 16
_R = 128
_S = 2048
_TAB = 2 * _R + 1
_G = 16
_WCOLS = 4096
_BAND = 512
_NBLK = _S // 8
_NGRP = _NBLK // 16

_mesh = plsc.VectorSubcoreMesh(core_axis_name="c", subcore_axis_name="s")


@functools.partial(
    pl.kernel,
    mesh=_mesh,
    out_type=jax.ShapeDtypeStruct((_NUM_HEADS, _S, _S), jnp.float32),
    compiler_params=pltpu.CompilerParams(needs_layout_passes=False),
    scratch_types=(
        [pltpu.VMEM((_TAB * _NUM_HEADS,), jnp.float32)]
        + [pltpu.VMEM((_WCOLS,), jnp.float32)] * _G
        + [
            pltpu.VMEM((8, _S), jnp.float32),
            pltpu.VMEM((8, _S), jnp.float32),
            pltpu.VMEM((8, _BAND), jnp.float32),
            pltpu.VMEM((8, _BAND), jnp.float32),
            pltpu.VMEM((8, _BAND), jnp.float32),
            pltpu.VMEM((8, _BAND), jnp.float32),
            pltpu.SemaphoreType.DMA,
            pltpu.SemaphoreType.DMA,
        ]
    ),
)
def _rpe_sc(emb_hbm, out_hbm, emb_v, *scratch):
    w_refs = scratch[:_G]
    lo_v, hi_v = scratch[_G], scratch[_G + 1]
    bands = scratch[_G + 2:_G + 6]
    sem_b, sem_c = scratch[_G + 6], scratch[_G + 7]
    wid = lax.axis_index("s") * 2 + lax.axis_index("c")
    h = wid % _NUM_HEADS
    half = wid // _NUM_HEADS

    pltpu.sync_copy(emb_hbm, emb_v)

    lanes = lax.iota(jnp.int32, 16)
    hvec = jnp.full((16,), h, dtype=jnp.int32)

    def build_w(c, carry):
        base = c * 16
        for r in range(_G):
            m = base + lanes + r
            idx = jnp.clip(m - (_S - 1 - _R), 0, _TAB - 1) * _NUM_HEADS + hvec
            w_refs[r][pl.ds(base, 16)] = plsc.load_gather(emb_v, [idx])
        return carry

    lax.fori_loop(0, _WCOLS // 16, build_w, 0)

    lo_vec = plsc.load_gather(emb_v, [hvec])
    hi_vec = plsc.load_gather(emb_v, [(_TAB - 1) * _NUM_HEADS + hvec])

    def fill_const(c, carry):
        for r_lo in range(8):
            lo_v[r_lo, pl.ds(c * 16, 16)] = lo_vec
            hi_v[r_lo, pl.ds(c * 16, 16)] = hi_vec
        return carry

    lax.fori_loop(0, _S // 16, fill_const, 0)

    def make_band_builder(band_ref, b):
        def build_band(blk, col0):
            s_base = _S - 1 - 8 * blk
            rs = [(15 - 8 * b - r_lo) % _G for r_lo in range(8)]
            qs = [
                pl.multiple_of(s_base - r_lo - rs[r_lo] + col0, _G)
                for r_lo in range(8)
            ]

            @plsc.parallel_loop(0, _BAND // 16, unroll=2)
            def chunk(cc):
                base = cc * 16
                for r_lo in range(8):
                    src = pl.multiple_of(qs[r_lo] + base, _G)
                    band_ref[r_lo, pl.ds(base, 16)] = w_refs[rs[r_lo]][pl.ds(src, 16)]

        return build_band

    builders = tuple(
        make_band_builder(bands[b], b % 2) for b in range(4)
    )

    for g in range(_NGRP):
        tb = min(12, max(0, g - 1))
        left = 128 * tb
        right0 = left + _BAND
        rlen = _S - right0

        @pl.when((g // (_NGRP // 2)) == half)
        def _(g=g, tb=tb, left=left, right0=right0, rlen=rlen):
            def quad(qq, carry):
                for b in range(4):
                    blk = g * 16 + qq * 4 + b
                    i_b = pl.multiple_of(8 * blk, 8)
                    if g % (_NGRP // 2) == 0:
                        @pl.when(qq > 0)
                        def _():
                            pltpu.make_async_copy(
                                bands[b],
                                out_hbm.at[h, pl.ds(0, 8), pl.ds(left, _BAND)],
                                sem_b,
                            ).wait()
                    else:
                        pltpu.make_async_copy(
                            bands[b],
                            out_hbm.at[h, pl.ds(0, 8), pl.ds(left, _BAND)],
                            sem_b,
                        ).wait()
                    builders[b](blk, left)
                    if left > 0:
                        pltpu.make_async_copy(
                            lo_v.at[:, pl.ds(0, left)],
                            out_hbm.at[h, pl.ds(i_b, 8), pl.ds(0, left)],
                            sem_c,
                        ).start()
                    pltpu.make_async_copy(
                        bands[b],
                        out_hbm.at[h, pl.ds(i_b, 8), pl.ds(left, _BAND)],
                        sem_b,
                    ).start()
                    if rlen > 0:
                        pltpu.make_async_copy(
                            hi_v.at[:, pl.ds(0, rlen)],
                            out_hbm.at[h, pl.ds(i_b, 8), pl.ds(right0, rlen)],
                            sem_c,
                        ).start()
                return carry

            lax.fori_loop(0, 4, quad, 0)

            def drain(tt, carry):
                i_b = pl.multiple_of(8 * (g * 16 + tt), 8)
                if left > 0:
                    pltpu.make_async_copy(
                        lo_v.at[:, pl.ds(0, left)],
                        out_hbm.at[h, pl.ds(i_b, 8), pl.ds(0, left)],
                        sem_c,
                    ).wait()
                if rlen > 0:
                    pltpu.make_async_copy(
                        hi_v.at[:, pl.ds(0, rlen)],
                        out_hbm.at[h, pl.ds(i_b, 8), pl.ds(right0, rlen)],
                        sem_c,
                    ).wait()
                return carry

            if left > 0 or rlen > 0:
                lax.fori_loop(0, 16, drain, 0)

    for b in range(4):
        pltpu.make_async_copy(
            bands[b], out_hbm.at[h, pl.ds(0, 8), pl.ds(0, _BAND)], sem_b
        ).wait()


def kernel(seq_len, emb):
    del seq_len
    return _rpe_sc(emb.reshape(_TAB * _NUM_HEADS))


# speedup vs baseline: 131.2412x; 1.0935x over previous
import functools

import jax
import jax.numpy as jnp
from jax import lax
from jax.experimental import pallas as pl
from jax.experimental.pallas import tpu as pltpu
from jax.experimental.pallas import tpu_sc as plsc

_NUM_HEADS = 16
_R = 128
_S = 2048
_TAB = 2 * _R + 1
_G = 16
_WCOLS = 4096
_BAND = 512
_NBLK = _S // 8
_NGRP = _NBLK // 16

_mesh = plsc.VectorSubcoreMesh(core_axis_name="c", subcore_axis_name="s")


@functools.partial(
    pl.kernel,
    mesh=_mesh,
    out_type=jax.ShapeDtypeStruct((_NUM_HEADS, _S, _S), jnp.float32),
    compiler_params=pltpu.CompilerParams(needs_layout_passes=False),
    scratch_types=(
        [pltpu.VMEM((_TAB * _NUM_HEADS,), jnp.float32)]
        + [pltpu.VMEM((_WCOLS,), jnp.float32)] * _G
        + [
            pltpu.VMEM((8, _S), jnp.float32),
            pltpu.VMEM((8, _S), jnp.float32),
            pltpu.VMEM((8, _BAND), jnp.float32),
            pltpu.VMEM((8, _BAND), jnp.float32),
            pltpu.VMEM((8, _BAND), jnp.float32),
            pltpu.VMEM((8, _BAND), jnp.float32),
            pltpu.SemaphoreType.DMA,
            pltpu.SemaphoreType.DMA,
        ]
    ),
)
def _rpe_sc(emb_hbm, out_hbm, emb_v, *scratch):
    w_refs = scratch[:_G]
    lo_v, hi_v = scratch[_G], scratch[_G + 1]
    bands = scratch[_G + 2:_G + 6]
    sem_b, sem_c = scratch[_G + 6], scratch[_G + 7]
    wid = lax.axis_index("s") * 2 + lax.axis_index("c")
    h = wid % _NUM_HEADS
    half = wid // _NUM_HEADS

    pltpu.sync_copy(emb_hbm, emb_v)

    lanes = lax.iota(jnp.int32, 16)
    hvec = jnp.full((16,), h, dtype=jnp.int32)

    @plsc.parallel_loop(0, _WCOLS // 16, unroll=2)
    def build_w(c):
        base = c * 16
        for r in range(_G):
            m = base + lanes + r
            idx = jnp.clip(m - (_S - 1 - _R), 0, _TAB - 1) * _NUM_HEADS + hvec
            w_refs[r][pl.ds(base, 16)] = plsc.load_gather(emb_v, [idx])

    lo_vec = plsc.load_gather(emb_v, [hvec])
    hi_vec = plsc.load_gather(emb_v, [(_TAB - 1) * _NUM_HEADS + hvec])

    @plsc.parallel_loop(0, _S // 16, unroll=2)
    def fill_const(c):
        for r_lo in range(8):
            lo_v[r_lo, pl.ds(c * 16, 16)] = lo_vec
            hi_v[r_lo, pl.ds(c * 16, 16)] = hi_vec

    def make_band_builder(band_ref, b):
        def build_band(blk, col0):
            s_base = _S - 1 - 8 * blk
            rs = [(15 - 8 * b - r_lo) % _G for r_lo in range(8)]
            qs = [
                pl.multiple_of(s_base - r_lo - rs[r_lo] + col0, _G)
                for r_lo in range(8)
            ]

            @plsc.parallel_loop(0, _BAND // 16, unroll=2)
            def chunk(cc):
                base = cc * 16
                for r_lo in range(8):
                    src = pl.multiple_of(qs[r_lo] + base, _G)
                    band_ref[r_lo, pl.ds(base, 16)] = w_refs[rs[r_lo]][pl.ds(src, 16)]

        return build_band

    builders = tuple(
        make_band_builder(bands[b], b % 2) for b in range(4)
    )

    for g in range(_NGRP):
        tb = min(12, max(0, g - 1))
        left = 128 * tb
        right0 = left + _BAND
        rlen = _S - right0

        @pl.when((g // (_NGRP // 2)) == half)
        def _(g=g, tb=tb, left=left, right0=right0, rlen=rlen):
            def quad(qq, carry):
                for b in range(4):
                    blk = g * 16 + qq * 4 + b
                    i_b = pl.multiple_of(8 * blk, 8)
                    if g % (_NGRP // 2) == 0:
                        @pl.when(qq > 0)
                        def _():
                            pltpu.make_async_copy(
                                bands[b],
                                out_hbm.at[h, pl.ds(0, 8), pl.ds(left, _BAND)],
                                sem_b,
                            ).wait()
                    else:
                        pltpu.make_async_copy(
                            bands[b],
                            out_hbm.at[h, pl.ds(0, 8), pl.ds(left, _BAND)],
                            sem_b,
                        ).wait()
                    builders[b](blk, left)
                    if left > 0:
                        pltpu.make_async_copy(
                            lo_v.at[:, pl.ds(0, left)],
                            out_hbm.at[h, pl.ds(i_b, 8), pl.ds(0, left)],
                            sem_c,
                        ).start()
                    pltpu.make_async_copy(
                        bands[b],
                        out_hbm.at[h, pl.ds(i_b, 8), pl.ds(left, _BAND)],
                        sem_b,
                    ).start()
                    if rlen > 0:
                        pltpu.make_async_copy(
                            hi_v.at[:, pl.ds(0, rlen)],
                            out_hbm.at[h, pl.ds(i_b, 8), pl.ds(right0, rlen)],
                            sem_c,
                        ).start()
                return carry

            lax.fori_loop(0, 4, quad, 0)

            def drain(tt, carry):
                i_b = pl.multiple_of(8 * (g * 16 + tt), 8)
                if left > 0:
                    pltpu.make_async_copy(
                        lo_v.at[:, pl.ds(0, left)],
                        out_hbm.at[h, pl.ds(i_b, 8), pl.ds(0, left)],
                        sem_c,
                    ).wait()
                if rlen > 0:
                    pltpu.make_async_copy(
                        hi_v.at[:, pl.ds(0, rlen)],
                        out_hbm.at[h, pl.ds(i_b, 8), pl.ds(right0, rlen)],
                        sem_c,
                    ).wait()
                return carry

            if left > 0 or rlen > 0:
                lax.fori_loop(0, 16, drain, 0)

    for b in range(4):
        pltpu.make_async_copy(
            bands[b], out_hbm.at[h, pl.ds(0, 8), pl.ds(0, _BAND)], sem_b
        ).wait()


def kernel(seq_len, emb):
    del seq_len
    return _rpe_sc(emb.reshape(_TAB * _NUM_HEADS))


# speedup vs baseline: 137.2457x; 1.0458x over previous
import functools

import jax
import jax.numpy as jnp
from jax import lax
from jax.experimental import pallas as pl
from jax.experimental.pallas import tpu as pltpu
from jax.experimental.pallas import tpu_sc as plsc

_NUM_HEADS = 16
_R = 128
_S = 2048
_TAB = 2 * _R + 1
_G = 16
_WCOLS = 4096
_BAND = 512
_NBLK = _S // 8
_NGRP = _NBLK // 16

_mesh = plsc.VectorSubcoreMesh(core_axis_name="c", subcore_axis_name="s")


@functools.partial(
    pl.kernel,
    mesh=_mesh,
    out_type=jax.ShapeDtypeStruct((_NUM_HEADS, _S, _S), jnp.float32),
    compiler_params=pltpu.CompilerParams(needs_layout_passes=False),
    scratch_types=(
        [pltpu.VMEM((_TAB * _NUM_HEADS,), jnp.float32)]
        + [pltpu.VMEM((_WCOLS,), jnp.float32)]
        + [
            pltpu.VMEM((8, _S), jnp.float32),
            pltpu.VMEM((8, _S), jnp.float32),
            pltpu.VMEM((8, _BAND), jnp.float32),
            pltpu.VMEM((8, _BAND), jnp.float32),
            pltpu.VMEM((8, _BAND), jnp.float32),
            pltpu.VMEM((8, _BAND), jnp.float32),
            pltpu.SemaphoreType.DMA,
            pltpu.SemaphoreType.DMA,
        ]
    ),
)
def _rpe_sc(emb_hbm, out_hbm, emb_v, *scratch):
    v_ref = scratch[0]
    lo_v, hi_v = scratch[1], scratch[2]
    bands = scratch[3:7]
    sem_b, sem_c = scratch[7], scratch[8]
    wid = lax.axis_index("s") * 2 + lax.axis_index("c")
    h = wid % _NUM_HEADS
    half = wid // _NUM_HEADS

    pltpu.sync_copy(emb_hbm, emb_v)

    lanes = lax.iota(jnp.int32, 16)
    hvec = jnp.full((16,), h, dtype=jnp.int32)

    @plsc.parallel_loop(0, _WCOLS // 16, unroll=2)
    def build_w(c):
        base = c * 16
        m = base + lanes
        idx = jnp.clip(m - (_S - 1 - _R), 0, _TAB - 1) * _NUM_HEADS + hvec
        v_ref[pl.ds(base, 16)] = plsc.load_gather(emb_v, [idx])

    lo_vec = plsc.load_gather(emb_v, [hvec])
    hi_vec = plsc.load_gather(emb_v, [(_TAB - 1) * _NUM_HEADS + hvec])

    @plsc.parallel_loop(0, _S // 16, unroll=2)
    def fill_const(c):
        for r_lo in range(8):
            lo_v[r_lo, pl.ds(c * 16, 16)] = lo_vec
            hi_v[r_lo, pl.ds(c * 16, 16)] = hi_vec

    def make_band_builder(band_ref):
        def build_band(blk, col0):
            s_base = _S - 1 - 8 * blk
            qs = [s_base - r_lo + col0 for r_lo in range(8)]

            @plsc.parallel_loop(0, _BAND // 16, unroll=2)
            def chunk(cc):
                base = cc * 16
                for r_lo in range(8):
                    band_ref[r_lo, pl.ds(base, 16)] = v_ref[pl.ds(qs[r_lo] + base, 16)]

        return build_band

    builders = tuple(make_band_builder(bands[b]) for b in range(4))

    for g in range(_NGRP):
        tb = min(12, max(0, g - 1))
        left = 128 * tb
        right0 = left + _BAND
        rlen = _S - right0

        @pl.when((g // (_NGRP // 2)) == half)
        def _(g=g, tb=tb, left=left, right0=right0, rlen=rlen):
            def quad(qq, carry):
                for b in range(4):
                    blk = g * 16 + qq * 4 + b
                    i_b = pl.multiple_of(8 * blk, 8)
                    if g % (_NGRP // 2) == 0:
                        @pl.when(qq > 0)
                        def _():
                            pltpu.make_async_copy(
                                bands[b],
                                out_hbm.at[h, pl.ds(0, 8), pl.ds(left, _BAND)],
                                sem_b,
                            ).wait()
                    else:
                        pltpu.make_async_copy(
                            bands[b],
                            out_hbm.at[h, pl.ds(0, 8), pl.ds(left, _BAND)],
                            sem_b,
                        ).wait()
                    builders[b](blk, left)
                    if left > 0:
                        pltpu.make_async_copy(
                            lo_v.at[:, pl.ds(0, left)],
                            out_hbm.at[h, pl.ds(i_b, 8), pl.ds(0, left)],
                            sem_c,
                        ).start()
                    pltpu.make_async_copy(
                        bands[b],
                        out_hbm.at[h, pl.ds(i_b, 8), pl.ds(left, _BAND)],
                        sem_b,
                    ).start()
                    if rlen > 0:
                        pltpu.make_async_copy(
                            hi_v.at[:, pl.ds(0, rlen)],
                            out_hbm.at[h, pl.ds(i_b, 8), pl.ds(right0, rlen)],
                            sem_c,
                        ).start()
                return carry

            lax.fori_loop(0, 4, quad, 0)

            def drain(tt, carry):
                i_b = pl.multiple_of(8 * (g * 16 + tt), 8)
                if left > 0:
                    pltpu.make_async_copy(
                        lo_v.at[:, pl.ds(0, left)],
                        out_hbm.at[h, pl.ds(i_b, 8), pl.ds(0, left)],
                        sem_c,
                    ).wait()
                if rlen > 0:
                    pltpu.make_async_copy(
                        hi_v.at[:, pl.ds(0, rlen)],
                        out_hbm.at[h, pl.ds(i_b, 8), pl.ds(right0, rlen)],
                        sem_c,
                    ).wait()
                return carry

            if left > 0 or rlen > 0:
                lax.fori_loop(0, 16, drain, 0)

    for b in range(4):
        pltpu.make_async_copy(
            bands[b], out_hbm.at[h, pl.ds(0, 8), pl.ds(0, _BAND)], sem_b
        ).wait()


def kernel(seq_len, emb):
    del seq_len
    return _rpe_sc(emb.reshape(_TAB * _NUM_HEADS))
